# Initial kernel scaffold; baseline (speedup 1.0000x reference)
import jax
import jax.numpy as jnp
from jax.experimental import pallas as pl


def kernel(x, edge_index, c, W, b):
    raise NotImplementedError("write your pallas kernel here")


# speedup vs baseline: 3.7063x; 3.7063x over previous
import functools

import jax
import jax.numpy as jnp
from jax import lax
from jax.experimental import pallas as pl
from jax.experimental.pallas import tpu as pltpu
from jax.experimental.pallas import tpu_sc as plsc

EPS = 1e-15
MAX_TANH_ARG = 1.0 - 1e-5

NC = 2
NS = 16
CH = 128
CW = 8
HD = 64


def _atanh(a):
  return 0.5 * jnp.log((1.0 + a) / (1.0 - a))


def _pre_body(c_ref, x_ref, o_ref):
  sqrt_c = jnp.sqrt(c_ref[0, 0])
  x = x_ref[...]
  norm = jnp.maximum(jnp.sqrt(jnp.sum(x * x, axis=-1, keepdims=True)), EPS)
  arg = jnp.minimum(sqrt_c * norm, MAX_TANH_ARG)
  o_ref[...] = _atanh(arg) * x / (sqrt_c * norm)


def _log_map_tc(x, c, block):
  n, d = x.shape
  grid = n // block
  return pl.pallas_call(
      _pre_body,
      grid=(grid,),
      in_specs=[
          pl.BlockSpec((1, 1), lambda i: (0, 0)),
          pl.BlockSpec((block, d), lambda i: (i, 0)),
      ],
      out_specs=pl.BlockSpec((block, d), lambda i: (i, 0)),
      out_shape=jax.ShapeDtypeStruct((n, d), jnp.float32),
  )(c.reshape(1, 1), x)


def _make_sc_agg(npad, nchunk):
  mesh = plsc.VectorSubcoreMesh(core_axis_name="c", subcore_axis_name="s")
  pr = npad // NS
  tail = pr - 4 * CH

  @functools.partial(
      pl.kernel,
      out_type=jax.ShapeDtypeStruct((NC, NS, pr, HD), jnp.float32),
      mesh=mesh,
      compiler_params=pltpu.CompilerParams(use_tc_tiling_on_sc=False),
      scratch_types=[
          pltpu.VMEM((nchunk, CH), jnp.int32),
          pltpu.VMEM((nchunk, CH), jnp.int32),
          pltpu.VMEM((CH, HD), jnp.float32),
          pltpu.VMEM_SHARED((npad, HD), jnp.float32),
          pltpu.SemaphoreType.DMA,
      ],
  )
  def sc_agg(v2_hbm, src_hbm, dst_hbm, z64_hbm,
             part_out,
             src_v, dst_v, rows_v, agg_sh, sem):
    cid = lax.axis_index("c")
    sid = lax.axis_index("s")
    pltpu.sync_copy(src_hbm.at[cid, sid], src_v)
    pltpu.sync_copy(dst_hbm.at[sid], dst_v)
    pltpu.sync_copy(z64_hbm, rows_v)

    def zinit(t, carry):
      pltpu.sync_copy(rows_v, agg_sh.at[pl.ds(sid * pr + t * CH, CH)])
      return carry

    lax.fori_loop(0, 4, zinit, 0)
    pltpu.sync_copy(rows_v.at[pl.ds(0, tail)],
                    agg_sh.at[pl.ds(sid * pr + 4 * CH, tail)])
    plsc.subcore_barrier()

    def body(j, carry):
      pltpu.async_copy(v2_hbm.at[src_v.at[j]], rows_v, sem).wait()
      pltpu.sync_copy(rows_v, agg_sh.at[dst_v.at[j]], add=True)
      return carry

    lax.fori_loop(0, nchunk, body, 0)
    plsc.subcore_barrier()

    def cout(t, carry):
      pltpu.sync_copy(agg_sh.at[pl.ds(sid * pr + t * CH, CH)], rows_v)
      pltpu.sync_copy(rows_v, part_out.at[cid, sid, pl.ds(t * CH, CH)])
      return carry

    lax.fori_loop(0, 4, cout, 0)
    pltpu.sync_copy(agg_sh.at[pl.ds(sid * pr + 4 * CH, tail)],
                    rows_v.at[pl.ds(0, tail)])
    pltpu.sync_copy(rows_v.at[pl.ds(0, tail)],
                    part_out.at[cid, sid, pl.ds(4 * CH, tail)])

  return sc_agg


def _make_sc_cnt(npad, nchunkb):
  mesh = plsc.VectorSubcoreMesh(core_axis_name="c", subcore_axis_name="s")
  pr = npad // NS
  tail = pr - 4 * CH

  @functools.partial(
      pl.kernel,
      out_type=jax.ShapeDtypeStruct((NC, NS, pr, CW), jnp.float32),
      mesh=mesh,
      compiler_params=pltpu.CompilerParams(use_tc_tiling_on_sc=False),
      scratch_types=[
          pltpu.VMEM((nchunkb, CH), jnp.int32),
          pltpu.VMEM((CH, CW), jnp.float32),
          pltpu.VMEM((CH, CW), jnp.float32),
          pltpu.VMEM_SHARED((npad, CW), jnp.float32),
      ],
  )
  def sc_cnt(dst_hbm, z8_hbm, o8_hbm,
             cnt_out,
             dst_v, zbuf, obuf, cnt_sh):
    cid = lax.axis_index("c")
    sid = lax.axis_index("s")
    pltpu.sync_copy(dst_hbm.at[cid, sid], dst_v)
    pltpu.sync_copy(z8_hbm, zbuf)
    pltpu.sync_copy(o8_hbm, obuf)

    def zinit(t, carry):
      pltpu.sync_copy(zbuf, cnt_sh.at[pl.ds(sid * pr + t * CH, CH)])
      return carry

    lax.fori_loop(0, 4, zinit, 0)
    pltpu.sync_copy(zbuf.at[pl.ds(0, tail)],
                    cnt_sh.at[pl.ds(sid * pr + 4 * CH, tail)])
    plsc.subcore_barrier()

    def body(j, carry):
      pltpu.sync_copy(obuf, cnt_sh.at[dst_v.at[j]], add=True)
      return carry

    lax.fori_loop(0, nchunkb, body, 0)
    plsc.subcore_barrier()

    def cout(t, carry):
      pltpu.sync_copy(cnt_sh.at[pl.ds(sid * pr + t * CH, CH)], zbuf)
      pltpu.sync_copy(zbuf, cnt_out.at[cid, sid, pl.ds(t * CH, CH)])
      return carry

    lax.fori_loop(0, 4, cout, 0)
    pltpu.sync_copy(cnt_sh.at[pl.ds(sid * pr + 4 * CH, tail)],
                    zbuf.at[pl.ds(0, tail)])
    pltpu.sync_copy(zbuf.at[pl.ds(0, tail)],
                    cnt_out.at[cid, sid, pl.ds(4 * CH, tail)])

  return sc_cnt


def _post_body(c_ref, v_ref, p0_ref, p1_ref, c0_ref, c1_ref, wt_ref, b_ref,
               o_ref):
  sqrt_c = jnp.sqrt(c_ref[0, 0])
  v = v_ref[...]
  agg = jnp.concatenate([p0_ref[...], p1_ref[...]], axis=1) + v
  cnt = c0_ref[:, :1] + c1_ref[:, :1] + 1.0
  va = agg / jnp.maximum(cnt, 1.0)
  r1 = jnp.maximum(jnp.sqrt(jnp.sum(va * va, axis=-1, keepdims=True)), EPS)
  vh = jnp.tanh(sqrt_c * r1) * va / (sqrt_c * r1)
  r2 = jnp.maximum(jnp.sqrt(jnp.sum(vh * vh, axis=-1, keepdims=True)), EPS)
  a2 = jnp.minimum(sqrt_c * r2, MAX_TANH_ARG)
  vv = _atanh(a2) * vh / (sqrt_c * r2)
  out = jnp.dot(vv, wt_ref[...], preferred_element_type=jnp.float32)
  out = out + b_ref[...]
  r3 = jnp.maximum(jnp.sqrt(jnp.sum(out * out, axis=-1, keepdims=True)), EPS)
  o_ref[...] = jnp.tanh(sqrt_c * r3) * out / (sqrt_c * r3)


def _post_tc(v, part, cnt, c, Wt, b, block):
  n, d = v.shape
  od = Wt.shape[1]
  grid = n // block
  return pl.pallas_call(
      _post_body,
      grid=(grid,),
      in_specs=[
          pl.BlockSpec((1, 1), lambda i: (0, 0)),
          pl.BlockSpec((block, d), lambda i: (i, 0)),
          pl.BlockSpec((block, HD), lambda i: (i, 0)),
          pl.BlockSpec((block, HD), lambda i: (i, 0)),
          pl.BlockSpec((block, CW), lambda i: (i, 0)),
          pl.BlockSpec((block, CW), lambda i: (i, 0)),
          pl.BlockSpec((d, od), lambda i: (0, 0)),
          pl.BlockSpec((1, od), lambda i: (0, 0)),
      ],
      out_specs=pl.BlockSpec((block, od), lambda i: (i, 0)),
      out_shape=jax.ShapeDtypeStruct((n, od), jnp.float32),
  )(c.reshape(1, 1), v, part[0], part[1], cnt[0], cnt[1], Wt,
    b.reshape(1, od))


def kernel(x, edge_index, c, W, b):
  n, d = x.shape
  e = edge_index.shape[1]
  c = c.astype(jnp.float32)

  nchunk = ((-(-e // (NS * CH)) + 15) // 16) * 16
  epad = NS * CH * nchunk
  npad = ((n + 1 + 2 * NS - 1) // (2 * NS)) * 2 * NS

  src = edge_index[0].astype(jnp.int32)
  dst = edge_index[1].astype(jnp.int32)
  if epad > e:
    src = jnp.concatenate([src, jnp.zeros((epad - e,), jnp.int32)])
    dst = jnp.concatenate([dst, jnp.full((epad - e,), n, jnp.int32)])
  src3 = jnp.stack([2 * src + c_ for c_ in range(NC)]
                   ).reshape(NC, NS, nchunk, CH)
  dst3 = dst.reshape(NS, nchunk, CH)
  dstb = dst.reshape(NC, NS, nchunk // 2, CH)

  z64 = jnp.zeros((CH, HD), jnp.float32)
  z8 = jnp.zeros((CH, CW), jnp.float32)
  o8 = jnp.ones((CH, CW), jnp.float32)

  v = _log_map_tc(x, c, block=1000)
  v2 = v.reshape(2 * n, HD)
  part = _make_sc_agg(npad, nchunk)(v2, src3, dst3, z64)
  cnt = _make_sc_cnt(npad, nchunk // 2)(dstb, z8, o8)
  part = part.reshape(NC, npad, HD)
  cnt = cnt.reshape(NC, npad, CW)
  return _post_tc(v, part, cnt, c, W.T, b, block=1000)


# speedup vs baseline: 4.1419x; 1.1175x over previous
import functools

import jax
import jax.numpy as jnp
from jax import lax
from jax.experimental import pallas as pl
from jax.experimental.pallas import tpu as pltpu
from jax.experimental.pallas import tpu_sc as plsc

EPS = 1e-15
MAX_TANH_ARG = 1.0 - 1e-5

NC = 2
NS = 16
CH = 128
CW = 8
HD = 64


def _atanh(a):
  return 0.5 * jnp.log((1.0 + a) / (1.0 - a))


def _pre_body(c_ref, x_ref, o_ref):
  sqrt_c = jnp.sqrt(c_ref[0, 0])
  x = x_ref[...]
  norm = jnp.maximum(jnp.sqrt(jnp.sum(x * x, axis=-1, keepdims=True)), EPS)
  arg = jnp.minimum(sqrt_c * norm, MAX_TANH_ARG)
  o_ref[...] = _atanh(arg) * x / (sqrt_c * norm)


def _log_map_tc(x, c, block):
  n, d = x.shape
  grid = n // block
  return pl.pallas_call(
      _pre_body,
      grid=(grid,),
      in_specs=[
          pl.BlockSpec((1, 1), lambda i: (0, 0)),
          pl.BlockSpec((block, d), lambda i: (i, 0)),
      ],
      out_specs=pl.BlockSpec((block, d), lambda i: (i, 0)),
      out_shape=jax.ShapeDtypeStruct((n, d), jnp.float32),
  )(c.reshape(1, 1), x)


def _make_sc_agg(npad, nchunk):
  mesh = plsc.VectorSubcoreMesh(core_axis_name="c", subcore_axis_name="s")
  pr = npad // NS
  tail = pr - 4 * CH

  @functools.partial(
      pl.kernel,
      out_type=jax.ShapeDtypeStruct((NC, NS, pr, HD), jnp.float32),
      mesh=mesh,
      compiler_params=pltpu.CompilerParams(use_tc_tiling_on_sc=False),
      scratch_types=[
          pltpu.VMEM((nchunk, CH), jnp.int32),
          pltpu.VMEM((nchunk, CH), jnp.int32),
          pltpu.VMEM((CH, HD), jnp.float32),
          pltpu.VMEM((CH, HD), jnp.float32),
          pltpu.VMEM_SHARED((npad, HD), jnp.float32),
          pltpu.SemaphoreType.DMA,
          pltpu.SemaphoreType.DMA,
      ],
  )
  def sc_agg(v2_hbm, src_hbm, dst_hbm, z64_hbm,
             part_out,
             src_v, dst_v, rows_v, rows_b, agg_sh, sem, sem_b):
    cid = lax.axis_index("c")
    sid = lax.axis_index("s")
    pltpu.sync_copy(src_hbm.at[cid, sid], src_v)
    pltpu.sync_copy(dst_hbm.at[sid], dst_v)
    pltpu.sync_copy(z64_hbm, rows_v)

    def zinit(t, carry):
      pltpu.sync_copy(rows_v, agg_sh.at[pl.ds(sid * pr + t * CH, CH)])
      return carry

    lax.fori_loop(0, 4, zinit, 0)
    pltpu.sync_copy(rows_v.at[pl.ds(0, tail)],
                    agg_sh.at[pl.ds(sid * pr + 4 * CH, tail)])
    plsc.subcore_barrier()

    pltpu.async_copy(v2_hbm.at[src_v.at[0]], rows_v, sem)

    def body(t, carry):
      j0 = 2 * t
      pltpu.async_copy(v2_hbm.at[src_v.at[j0 + 1]], rows_b, sem_b)
      pltpu.make_async_copy(v2_hbm.at[src_v.at[j0]], rows_v, sem).wait()
      pltpu.sync_copy(rows_v, agg_sh.at[dst_v.at[j0]], add=True)

      @pl.when(t + 1 < nchunk // 2)
      def _():
        pltpu.async_copy(v2_hbm.at[src_v.at[j0 + 2]], rows_v, sem)

      pltpu.make_async_copy(v2_hbm.at[src_v.at[j0 + 1]], rows_b,
                            sem_b).wait()
      pltpu.sync_copy(rows_b, agg_sh.at[dst_v.at[j0 + 1]], add=True)
      return carry

    lax.fori_loop(0, nchunk // 2, body, 0)
    plsc.subcore_barrier()

    def cout(t, carry):
      pltpu.sync_copy(agg_sh.at[pl.ds(sid * pr + t * CH, CH)], rows_v)
      pltpu.sync_copy(rows_v, part_out.at[cid, sid, pl.ds(t * CH, CH)])
      return carry

    lax.fori_loop(0, 4, cout, 0)
    pltpu.sync_copy(agg_sh.at[pl.ds(sid * pr + 4 * CH, tail)],
                    rows_v.at[pl.ds(0, tail)])
    pltpu.sync_copy(rows_v.at[pl.ds(0, tail)],
                    part_out.at[cid, sid, pl.ds(4 * CH, tail)])

  return sc_agg


def _make_sc_cnt(npad, nchunkb):
  mesh = plsc.VectorSubcoreMesh(core_axis_name="c", subcore_axis_name="s")
  pr = npad // NS
  tail = pr - 4 * CH

  @functools.partial(
      pl.kernel,
      out_type=jax.ShapeDtypeStruct((NC, NS, pr, CW), jnp.float32),
      mesh=mesh,
      compiler_params=pltpu.CompilerParams(use_tc_tiling_on_sc=False),
      scratch_types=[
          pltpu.VMEM((nchunkb, CH), jnp.int32),
          pltpu.VMEM((CH, CW), jnp.float32),
          pltpu.VMEM((CH, CW), jnp.float32),
          pltpu.VMEM_SHARED((npad, CW), jnp.float32),
      ],
  )
  def sc_cnt(dst_hbm, z8_hbm, o8_hbm,
             cnt_out,
             dst_v, zbuf, obuf, cnt_sh):
    cid = lax.axis_index("c")
    sid = lax.axis_index("s")
    pltpu.sync_copy(dst_hbm.at[cid, sid], dst_v)
    pltpu.sync_copy(z8_hbm, zbuf)
    pltpu.sync_copy(o8_hbm, obuf)

    def zinit(t, carry):
      pltpu.sync_copy(zbuf, cnt_sh.at[pl.ds(sid * pr + t * CH, CH)])
      return carry

    lax.fori_loop(0, 4, zinit, 0)
    pltpu.sync_copy(zbuf.at[pl.ds(0, tail)],
                    cnt_sh.at[pl.ds(sid * pr + 4 * CH, tail)])
    plsc.subcore_barrier()

    def body(j, carry):
      pltpu.sync_copy(obuf, cnt_sh.at[dst_v.at[j]], add=True)
      return carry

    lax.fori_loop(0, nchunkb, body, 0)
    plsc.subcore_barrier()

    def cout(t, carry):
      pltpu.sync_copy(cnt_sh.at[pl.ds(sid * pr + t * CH, CH)], zbuf)
      pltpu.sync_copy(zbuf, cnt_out.at[cid, sid, pl.ds(t * CH, CH)])
      return carry

    lax.fori_loop(0, 4, cout, 0)
    pltpu.sync_copy(cnt_sh.at[pl.ds(sid * pr + 4 * CH, tail)],
                    zbuf.at[pl.ds(0, tail)])
    pltpu.sync_copy(zbuf.at[pl.ds(0, tail)],
                    cnt_out.at[cid, sid, pl.ds(4 * CH, tail)])

  return sc_cnt


def _post_body(c_ref, v_ref, p0_ref, p1_ref, c0_ref, c1_ref, wt_ref, b_ref,
               o_ref):
  sqrt_c = jnp.sqrt(c_ref[0, 0])
  v = v_ref[...]
  agg = jnp.concatenate([p0_ref[...], p1_ref[...]], axis=1) + v
  cnt = c0_ref[:, :1] + c1_ref[:, :1] + 1.0
  va = agg / jnp.maximum(cnt, 1.0)
  r1 = jnp.maximum(jnp.sqrt(jnp.sum(va * va, axis=-1, keepdims=True)), EPS)
  vh = jnp.tanh(sqrt_c * r1) * va / (sqrt_c * r1)
  r2 = jnp.maximum(jnp.sqrt(jnp.sum(vh * vh, axis=-1, keepdims=True)), EPS)
  a2 = jnp.minimum(sqrt_c * r2, MAX_TANH_ARG)
  vv = _atanh(a2) * vh / (sqrt_c * r2)
  out = jnp.dot(vv, wt_ref[...], preferred_element_type=jnp.float32)
  out = out + b_ref[...]
  r3 = jnp.maximum(jnp.sqrt(jnp.sum(out * out, axis=-1, keepdims=True)), EPS)
  o_ref[...] = jnp.tanh(sqrt_c * r3) * out / (sqrt_c * r3)


def _post_tc(v, part, cnt, c, Wt, b, block):
  n, d = v.shape
  od = Wt.shape[1]
  grid = n // block
  return pl.pallas_call(
      _post_body,
      grid=(grid,),
      in_specs=[
          pl.BlockSpec((1, 1), lambda i: (0, 0)),
          pl.BlockSpec((block, d), lambda i: (i, 0)),
          pl.BlockSpec((block, HD), lambda i: (i, 0)),
          pl.BlockSpec((block, HD), lambda i: (i, 0)),
          pl.BlockSpec((block, CW), lambda i: (i, 0)),
          pl.BlockSpec((block, CW), lambda i: (i, 0)),
          pl.BlockSpec((d, od), lambda i: (0, 0)),
          pl.BlockSpec((1, od), lambda i: (0, 0)),
      ],
      out_specs=pl.BlockSpec((block, od), lambda i: (i, 0)),
      out_shape=jax.ShapeDtypeStruct((n, od), jnp.float32),
  )(c.reshape(1, 1), v, part[0], part[1], cnt[0], cnt[1], Wt,
    b.reshape(1, od))


def kernel(x, edge_index, c, W, b):
  n, d = x.shape
  e = edge_index.shape[1]
  c = c.astype(jnp.float32)

  nchunk = ((-(-e // (NS * CH)) + 15) // 16) * 16
  epad = NS * CH * nchunk
  npad = ((n + 1 + 2 * NS - 1) // (2 * NS)) * 2 * NS

  src = edge_index[0].astype(jnp.int32)
  dst = edge_index[1].astype(jnp.int32)
  if epad > e:
    src = jnp.concatenate([src, jnp.zeros((epad - e,), jnp.int32)])
    dst = jnp.concatenate([dst, jnp.full((epad - e,), n, jnp.int32)])
  src3 = jnp.stack([2 * src + c_ for c_ in range(NC)]
                   ).reshape(NC, NS, nchunk, CH)
  dst3 = dst.reshape(NS, nchunk, CH)
  dstb = dst.reshape(NC, NS, nchunk // 2, CH)

  z64 = jnp.zeros((CH, HD), jnp.float32)
  z8 = jnp.zeros((CH, CW), jnp.float32)
  o8 = jnp.ones((CH, CW), jnp.float32)

  v = _log_map_tc(x, c, block=1000)
  v2 = v.reshape(2 * n, HD)
  part = _make_sc_agg(npad, nchunk)(v2, src3, dst3, z64)
  cnt = _make_sc_cnt(npad, nchunk // 2)(dstb, z8, o8)
  part = part.reshape(NC, npad, HD)
  cnt = cnt.reshape(NC, npad, CW)
  return _post_tc(v, part, cnt, c, W.T, b, block=1000)
